# baseline (speedup 1.0000x reference)
import jax
import jax.numpy as jnp
from jax import lax
from jax.experimental import pallas as pl
from jax.experimental.pallas import tpu as pltpu
from jax.experimental.pallas import tpu_sc as plsc

B = 4
H = 128
W = 128
C = 192
HS = 64
NC = 2
NS = 16
NW = NC * NS
L = 16
SC_ROWS = H - HS
ROWS_PER_W = (B * SC_ROWS) // NW
RPC = 2
CHUNKS = ROWS_PER_W // RPC
UNROLL = 8


def _sc_kernel(sim_hbm, out_hbm, buf0, buf1, outb, sem0, sem1):
    cid = lax.axis_index("c")
    sid = lax.axis_index("s")
    wid = sid * NC + cid
    img = wid // (NW // B)
    row0 = HS + (wid % (NW // B)) * ROWS_PER_W

    def src(k):
        return sim_hbm.at[img, :, pl.ds(row0 + k * RPC, RPC), :]

    def compute(buf, k):
        for r in range(RPC):
            def group_body(j, _, r=r):
                base = j * (2 * L)
                neg = jnp.full((L,), -jnp.inf, jnp.float32)

                def step(i, carry):
                    a1, a2, b1, b2 = carry
                    for u in range(UNROLL):
                        c = i * UNROLL + u
                        va = buf[c, r, pl.ds(base, L)]
                        vb = buf[c, r, pl.ds(base + L, L)]
                        na1 = jnp.maximum(a1, va)
                        a2 = jnp.maximum(a2, jnp.minimum(a1, va))
                        a1 = na1
                        nb1 = jnp.maximum(b1, vb)
                        b2 = jnp.maximum(b2, jnp.minimum(b1, vb))
                        b1 = nb1
                    return a1, a2, b1, b2

                a1, a2, b1, b2 = lax.fori_loop(
                    0, C // UNROLL, step, (neg, neg, neg, neg))
                row = k * RPC + r
                outb[row, pl.ds(base, L)] = jnp.exp(1.0 - jnp.exp(a1 - a2))
                outb[row, pl.ds(base + L, L)] = jnp.exp(1.0 - jnp.exp(b1 - b2))
                return 0

            lax.fori_loop(0, W // (2 * L), group_body, 0)

    pltpu.async_copy(src(0), buf0, sem0)
    if CHUNKS > 1:
        pltpu.async_copy(src(1), buf1, sem1)

    def pair(g, _):
        k0 = 2 * g
        pltpu.make_async_copy(src(k0), buf0, sem0).wait()
        compute(buf0, k0)

        @pl.when(k0 + 2 < CHUNKS)
        def _():
            pltpu.async_copy(src(k0 + 2), buf0, sem0)

        k1 = k0 + 1
        pltpu.make_async_copy(src(k1), buf1, sem1).wait()
        compute(buf1, k1)

        @pl.when(k1 + 2 < CHUNKS)
        def _():
            pltpu.async_copy(src(k1 + 2), buf1, sem1)

        return 0

    lax.fori_loop(0, CHUNKS // 2, pair, 0)
    pltpu.sync_copy(outb, out_hbm.at[img, pl.ds(row0 - HS, ROWS_PER_W), :])


TC_ROWS = 32
TC_UNROLL = 8
TC_CHAINS = 2


def _tc_kernel(sim_ref, out_ref):
    neg = jnp.full((TC_ROWS, W), -jnp.inf, jnp.float32)

    def step(i, carry):
        ms = list(carry)
        for u in range(TC_UNROLL):
            c = i * TC_UNROLL + u
            ch = u % TC_CHAINS
            m1, m2 = ms[2 * ch], ms[2 * ch + 1]
            v = sim_ref[0, c]
            nm1 = jnp.maximum(m1, v)
            m2 = jnp.maximum(m2, jnp.minimum(m1, v))
            ms[2 * ch], ms[2 * ch + 1] = nm1, m2
        return tuple(ms)

    ms = lax.fori_loop(0, C // TC_UNROLL, step, (neg,) * (2 * TC_CHAINS))
    m1, m2 = ms[0], ms[1]
    for ch in range(1, TC_CHAINS):
        b1, b2 = ms[2 * ch], ms[2 * ch + 1]
        nm1 = jnp.maximum(m1, b1)
        m2 = jnp.maximum(jnp.minimum(m1, b1), jnp.maximum(m2, b2))
        m1 = nm1
    out_ref[0] = jnp.exp(1.0 - jnp.exp(m1 - m2))


@jax.jit
def _run(sim_mat):
    mesh = plsc.VectorSubcoreMesh(core_axis_name="c", subcore_axis_name="s")
    sc_fn = pl.kernel(
        _sc_kernel,
        out_type=jax.ShapeDtypeStruct((B, SC_ROWS, W), jnp.float32),
        mesh=mesh,
        scratch_types=[
            pltpu.VMEM((C, RPC, W), jnp.float32),
            pltpu.VMEM((C, RPC, W), jnp.float32),
            pltpu.VMEM((ROWS_PER_W, W), jnp.float32),
            pltpu.SemaphoreType.DMA,
            pltpu.SemaphoreType.DMA,
        ],
    )
    sc_out = sc_fn(sim_mat)

    tc_fn = pl.pallas_call(
        _tc_kernel,
        grid=(B, HS // TC_ROWS),
        in_specs=[pl.BlockSpec((1, C, TC_ROWS, W), lambda b, i: (b, 0, i, 0))],
        out_specs=pl.BlockSpec((1, TC_ROWS, W), lambda b, i: (b, i, 0)),
        out_shape=jax.ShapeDtypeStruct((B, HS, W), jnp.float32),
    )
    tc_out = tc_fn(sim_mat)

    return jnp.concatenate([tc_out, sc_out], axis=1)


def kernel(sim_mat, x):
    del x
    return (_run(sim_mat),)


# speedup vs baseline: 1.1016x; 1.1016x over previous
import jax
import jax.numpy as jnp
from jax import lax
from jax.experimental import pallas as pl
from jax.experimental.pallas import tpu as pltpu
from jax.experimental.pallas import tpu_sc as plsc

B = 4
H = 128
W = 128
C = 192
HS = 96
NC = 2
NS = 16
NW = NC * NS
L = 16
SC_ROWS = H - HS
ROWS_PER_W = (B * SC_ROWS) // NW
RPC = 2
CHUNKS = ROWS_PER_W // RPC
UNROLL = 8


def _sc_kernel(sim_hbm, out_hbm, buf0, buf1, outb, sem0, sem1):
    cid = lax.axis_index("c")
    sid = lax.axis_index("s")
    wid = sid * NC + cid
    img = wid // (NW // B)
    row0 = HS + (wid % (NW // B)) * ROWS_PER_W

    def src(k):
        return sim_hbm.at[img, :, pl.ds(row0 + k * RPC, RPC), :]

    def compute(buf, k):
        for r in range(RPC):
            def group_body(j, _, r=r):
                base = j * (2 * L)
                neg = jnp.full((L,), -jnp.inf, jnp.float32)

                def step(i, carry):
                    a1, a2, b1, b2 = carry
                    for u in range(UNROLL):
                        c = i * UNROLL + u
                        va = buf[c, r, pl.ds(base, L)]
                        vb = buf[c, r, pl.ds(base + L, L)]
                        na1 = jnp.maximum(a1, va)
                        a2 = jnp.maximum(a2, jnp.minimum(a1, va))
                        a1 = na1
                        nb1 = jnp.maximum(b1, vb)
                        b2 = jnp.maximum(b2, jnp.minimum(b1, vb))
                        b1 = nb1
                    return a1, a2, b1, b2

                a1, a2, b1, b2 = lax.fori_loop(
                    0, C // UNROLL, step, (neg, neg, neg, neg))
                row = k * RPC + r
                outb[row, pl.ds(base, L)] = jnp.exp(1.0 - jnp.exp(a1 - a2))
                outb[row, pl.ds(base + L, L)] = jnp.exp(1.0 - jnp.exp(b1 - b2))
                return 0

            lax.fori_loop(0, W // (2 * L), group_body, 0)

    pltpu.async_copy(src(0), buf0, sem0)
    if CHUNKS > 1:
        pltpu.async_copy(src(1), buf1, sem1)

    def pair(g, _):
        k0 = 2 * g
        pltpu.make_async_copy(src(k0), buf0, sem0).wait()
        compute(buf0, k0)

        @pl.when(k0 + 2 < CHUNKS)
        def _():
            pltpu.async_copy(src(k0 + 2), buf0, sem0)

        k1 = k0 + 1
        pltpu.make_async_copy(src(k1), buf1, sem1).wait()
        compute(buf1, k1)

        @pl.when(k1 + 2 < CHUNKS)
        def _():
            pltpu.async_copy(src(k1 + 2), buf1, sem1)

        return 0

    lax.fori_loop(0, CHUNKS // 2, pair, 0)
    pltpu.sync_copy(outb, out_hbm.at[img, pl.ds(row0 - HS, ROWS_PER_W), :])


TC_ROWS = 48
TC_UNROLL = 8
TC_CHAINS = 2


def _tc_kernel(sim_ref, out_ref):
    neg = jnp.full((TC_ROWS, W), -jnp.inf, jnp.float32)

    def step(i, carry):
        ms = list(carry)
        for u in range(TC_UNROLL):
            c = i * TC_UNROLL + u
            ch = u % TC_CHAINS
            m1, m2 = ms[2 * ch], ms[2 * ch + 1]
            v = sim_ref[0, c]
            nm1 = jnp.maximum(m1, v)
            m2 = jnp.maximum(m2, jnp.minimum(m1, v))
            ms[2 * ch], ms[2 * ch + 1] = nm1, m2
        return tuple(ms)

    ms = lax.fori_loop(0, C // TC_UNROLL, step, (neg,) * (2 * TC_CHAINS))
    m1, m2 = ms[0], ms[1]
    for ch in range(1, TC_CHAINS):
        b1, b2 = ms[2 * ch], ms[2 * ch + 1]
        nm1 = jnp.maximum(m1, b1)
        m2 = jnp.maximum(jnp.minimum(m1, b1), jnp.maximum(m2, b2))
        m1 = nm1
    out_ref[0] = jnp.exp(1.0 - jnp.exp(m1 - m2))


@jax.jit
def _run(sim_mat):
    mesh = plsc.VectorSubcoreMesh(core_axis_name="c", subcore_axis_name="s")
    sc_fn = pl.kernel(
        _sc_kernel,
        out_type=jax.ShapeDtypeStruct((B, SC_ROWS, W), jnp.float32),
        mesh=mesh,
        scratch_types=[
            pltpu.VMEM((C, RPC, W), jnp.float32),
            pltpu.VMEM((C, RPC, W), jnp.float32),
            pltpu.VMEM((ROWS_PER_W, W), jnp.float32),
            pltpu.SemaphoreType.DMA,
            pltpu.SemaphoreType.DMA,
        ],
    )
    sc_out = sc_fn(sim_mat)

    tc_fn = pl.pallas_call(
        _tc_kernel,
        grid=(B, HS // TC_ROWS),
        in_specs=[pl.BlockSpec((1, C, TC_ROWS, W), lambda b, i: (b, 0, i, 0))],
        out_specs=pl.BlockSpec((1, TC_ROWS, W), lambda b, i: (b, i, 0)),
        out_shape=jax.ShapeDtypeStruct((B, HS, W), jnp.float32),
    )
    tc_out = tc_fn(sim_mat)

    return jnp.concatenate([tc_out, sc_out], axis=1)


def kernel(sim_mat, x):
    del x
    return (_run(sim_mat),)


# speedup vs baseline: 1.1052x; 1.0033x over previous
import jax
import jax.numpy as jnp
from jax import lax
from jax.experimental import pallas as pl
from jax.experimental.pallas import tpu as pltpu
from jax.experimental.pallas import tpu_sc as plsc

B = 4
H = 128
W = 128
C = 192
HS = 96
NC = 2
NS = 16
NW = NC * NS
L = 16
SC_ROWS = H - HS
ROWS_PER_W = (B * SC_ROWS) // NW
RPC = 2
CHUNKS = ROWS_PER_W // RPC
UNROLL = 16


def _sc_kernel(sim_hbm, out_hbm, buf0, buf1, outb, sem0, sem1):
    cid = lax.axis_index("c")
    sid = lax.axis_index("s")
    wid = sid * NC + cid
    img = wid // (NW // B)
    row0 = HS + (wid % (NW // B)) * ROWS_PER_W

    def src(k):
        return sim_hbm.at[img, :, pl.ds(row0 + k * RPC, RPC), :]

    def compute(buf, k):
        for r in range(RPC):
            def group_body(j, _, r=r):
                base = j * (2 * L)
                neg = jnp.full((L,), -jnp.inf, jnp.float32)

                def step(i, carry):
                    a1, a2, b1, b2 = carry
                    for u in range(UNROLL):
                        c = i * UNROLL + u
                        va = buf[c, r, pl.ds(base, L)]
                        vb = buf[c, r, pl.ds(base + L, L)]
                        na1 = jnp.maximum(a1, va)
                        a2 = jnp.maximum(a2, jnp.minimum(a1, va))
                        a1 = na1
                        nb1 = jnp.maximum(b1, vb)
                        b2 = jnp.maximum(b2, jnp.minimum(b1, vb))
                        b1 = nb1
                    return a1, a2, b1, b2

                a1, a2, b1, b2 = lax.fori_loop(
                    0, C // UNROLL, step, (neg, neg, neg, neg))
                row = k * RPC + r
                outb[row, pl.ds(base, L)] = jnp.exp(1.0 - jnp.exp(a1 - a2))
                outb[row, pl.ds(base + L, L)] = jnp.exp(1.0 - jnp.exp(b1 - b2))
                return 0

            lax.fori_loop(0, W // (2 * L), group_body, 0)

    bufs = (buf0, buf1)
    sems = (sem0, sem1)
    for k in range(min(2, CHUNKS)):
        pltpu.async_copy(src(k), bufs[k], sems[k])
    for k in range(CHUNKS):
        bb = k % 2
        pltpu.make_async_copy(src(k), bufs[bb], sems[bb]).wait()
        compute(bufs[bb], k)
        if k + 2 < CHUNKS:
            pltpu.async_copy(src(k + 2), bufs[bb], sems[bb])
    pltpu.sync_copy(outb, out_hbm.at[img, pl.ds(row0 - HS, ROWS_PER_W), :])


TC_ROWS = 48
TC_UNROLL = 8
TC_CHAINS = 2


def _tc_kernel(sim_ref, out_ref):
    neg = jnp.full((TC_ROWS, W), -jnp.inf, jnp.float32)

    def step(i, carry):
        ms = list(carry)
        for u in range(TC_UNROLL):
            c = i * TC_UNROLL + u
            ch = u % TC_CHAINS
            m1, m2 = ms[2 * ch], ms[2 * ch + 1]
            v = sim_ref[0, c]
            nm1 = jnp.maximum(m1, v)
            m2 = jnp.maximum(m2, jnp.minimum(m1, v))
            ms[2 * ch], ms[2 * ch + 1] = nm1, m2
        return tuple(ms)

    ms = lax.fori_loop(0, C // TC_UNROLL, step, (neg,) * (2 * TC_CHAINS))
    m1, m2 = ms[0], ms[1]
    for ch in range(1, TC_CHAINS):
        b1, b2 = ms[2 * ch], ms[2 * ch + 1]
        nm1 = jnp.maximum(m1, b1)
        m2 = jnp.maximum(jnp.minimum(m1, b1), jnp.maximum(m2, b2))
        m1 = nm1
    out_ref[0] = jnp.exp(1.0 - jnp.exp(m1 - m2))


@jax.jit
def _run(sim_mat):
    mesh = plsc.VectorSubcoreMesh(core_axis_name="c", subcore_axis_name="s")
    sc_fn = pl.kernel(
        _sc_kernel,
        out_type=jax.ShapeDtypeStruct((B, SC_ROWS, W), jnp.float32),
        mesh=mesh,
        scratch_types=[
            pltpu.VMEM((C, RPC, W), jnp.float32),
            pltpu.VMEM((C, RPC, W), jnp.float32),
            pltpu.VMEM((ROWS_PER_W, W), jnp.float32),
            pltpu.SemaphoreType.DMA,
            pltpu.SemaphoreType.DMA,
        ],
    )
    sc_out = sc_fn(sim_mat)

    tc_fn = pl.pallas_call(
        _tc_kernel,
        grid=(B, HS // TC_ROWS),
        in_specs=[pl.BlockSpec((1, C, TC_ROWS, W), lambda b, i: (b, 0, i, 0))],
        out_specs=pl.BlockSpec((1, TC_ROWS, W), lambda b, i: (b, i, 0)),
        out_shape=jax.ShapeDtypeStruct((B, HS, W), jnp.float32),
    )
    tc_out = tc_fn(sim_mat)

    return jnp.concatenate([tc_out, sc_out], axis=1)


def kernel(sim_mat, x):
    del x
    return (_run(sim_mat),)


# speedup vs baseline: 1.1140x; 1.0080x over previous
import jax
import jax.numpy as jnp
from jax import lax
from jax.experimental import pallas as pl
from jax.experimental.pallas import tpu as pltpu
from jax.experimental.pallas import tpu_sc as plsc

B = 4
H = 128
W = 128
C = 192
HS = 96
NC = 2
NS = 16
NW = NC * NS
L = 16
SC_ROWS = H - HS
ROWS_PER_W = (B * SC_ROWS) // NW
RPC = 2
CHUNKS = ROWS_PER_W // RPC
UNROLL = 16


def _sc_kernel(sim_hbm, out_hbm, buf0, buf1, outb, sem0, sem1):
    cid = lax.axis_index("c")
    sid = lax.axis_index("s")
    wid = sid * NC + cid
    img = wid // (NW // B)
    row0 = HS + (wid % (NW // B)) * ROWS_PER_W

    def src(k):
        return sim_hbm.at[img, :, pl.ds(row0 + k * RPC, RPC), :]

    def compute(buf, k):
        for r in range(RPC):
            def group_body(j, _, r=r):
                base = j * (2 * L)
                neg = jnp.full((L,), -jnp.inf, jnp.float32)

                def step(i, carry):
                    a1, a2, b1, b2 = carry
                    for u in range(UNROLL):
                        c = i * UNROLL + u
                        va = buf[c, r, pl.ds(base, L)]
                        vb = buf[c, r, pl.ds(base + L, L)]
                        na1 = jnp.maximum(a1, va)
                        a2 = jnp.maximum(a2, jnp.minimum(a1, va))
                        a1 = na1
                        nb1 = jnp.maximum(b1, vb)
                        b2 = jnp.maximum(b2, jnp.minimum(b1, vb))
                        b1 = nb1
                    return a1, a2, b1, b2

                a1, a2, b1, b2 = lax.fori_loop(
                    0, C // UNROLL, step, (neg, neg, neg, neg))
                row = k * RPC + r
                outb[row, pl.ds(base, L)] = jnp.exp(1.0 - jnp.exp(a1 - a2))
                outb[row, pl.ds(base + L, L)] = jnp.exp(1.0 - jnp.exp(b1 - b2))
                return 0

            lax.fori_loop(0, W // (2 * L), group_body, 0)

    bufs = (buf0, buf1)
    sems = (sem0, sem1)
    for k in range(min(2, CHUNKS)):
        pltpu.async_copy(src(k), bufs[k], sems[k])
    for k in range(CHUNKS):
        bb = k % 2
        pltpu.make_async_copy(src(k), bufs[bb], sems[bb]).wait()
        compute(bufs[bb], k)
        if k + 2 < CHUNKS:
            pltpu.async_copy(src(k + 2), bufs[bb], sems[bb])
    pltpu.sync_copy(outb, out_hbm.at[img, pl.ds(row0 - HS, ROWS_PER_W), :])


TC_ROWS = 16
TC_UNROLL = 8
TC_CHAINS = 2
TC_NBUF = 4
TC_BLOCKS = [(i, j) for i in range(B) for j in range(HS // TC_ROWS)]


def _tc_kernel(sim_hbm, out_hbm, b0, b1, b2, b3, outv, s0, s1, s2, s3, osem):
    bufs = (b0, b1, b2, b3)
    sems = (s0, s1, s2, s3)

    def src(t):
        img, band = TC_BLOCKS[t]
        return sim_hbm.at[img, :, pl.ds(band * TC_ROWS, TC_ROWS), :]

    def compute(buf, img, band):
        neg = jnp.full((TC_ROWS, W), -jnp.inf, jnp.float32)

        def step(i, carry):
            ms = list(carry)
            for u in range(TC_UNROLL):
                c = i * TC_UNROLL + u
                ch = u % TC_CHAINS
                m1, m2 = ms[2 * ch], ms[2 * ch + 1]
                v = buf[c]
                nm1 = jnp.maximum(m1, v)
                m2 = jnp.maximum(m2, jnp.minimum(m1, v))
                ms[2 * ch], ms[2 * ch + 1] = nm1, m2
            return tuple(ms)

        ms = lax.fori_loop(0, C // TC_UNROLL, step, (neg,) * (2 * TC_CHAINS))
        m1, m2 = ms[0], ms[1]
        for ch in range(1, TC_CHAINS):
            c1, c2 = ms[2 * ch], ms[2 * ch + 1]
            nm1 = jnp.maximum(m1, c1)
            m2 = jnp.maximum(jnp.minimum(m1, c1), jnp.maximum(m2, c2))
            m1 = nm1
        outv[img, pl.ds(band * TC_ROWS, TC_ROWS), :] = (
            jnp.exp(1.0 - jnp.exp(m1 - m2)))

    for t in range(TC_NBUF):
        pltpu.make_async_copy(src(t), bufs[t], sems[t]).start()
    for t, (img, band) in enumerate(TC_BLOCKS):
        bb = t % TC_NBUF
        pltpu.make_async_copy(src(t), bufs[bb], sems[bb]).wait()
        compute(bufs[bb], img, band)
        nxt = t + TC_NBUF
        if nxt < len(TC_BLOCKS):
            pltpu.make_async_copy(src(nxt), bufs[bb], sems[bb]).start()
    cp = pltpu.make_async_copy(outv, out_hbm, osem)
    cp.start()
    cp.wait()


@jax.jit
def _run(sim_mat):
    mesh = plsc.VectorSubcoreMesh(core_axis_name="c", subcore_axis_name="s")
    sc_fn = pl.kernel(
        _sc_kernel,
        out_type=jax.ShapeDtypeStruct((B, SC_ROWS, W), jnp.float32),
        mesh=mesh,
        scratch_types=[
            pltpu.VMEM((C, RPC, W), jnp.float32),
            pltpu.VMEM((C, RPC, W), jnp.float32),
            pltpu.VMEM((ROWS_PER_W, W), jnp.float32),
            pltpu.SemaphoreType.DMA,
            pltpu.SemaphoreType.DMA,
        ],
    )
    sc_out = sc_fn(sim_mat)

    tc_fn = pl.pallas_call(
        _tc_kernel,
        in_specs=[pl.BlockSpec(memory_space=pl.ANY)],
        out_specs=pl.BlockSpec(memory_space=pl.ANY),
        out_shape=jax.ShapeDtypeStruct((B, HS, W), jnp.float32),
        scratch_shapes=(
            [pltpu.VMEM((C, TC_ROWS, W), jnp.float32)] * TC_NBUF
            + [pltpu.VMEM((B, HS, W), jnp.float32)]
            + [pltpu.SemaphoreType.DMA] * (TC_NBUF + 1)
        ),
    )
    tc_out = tc_fn(sim_mat)

    return jnp.concatenate([tc_out, sc_out], axis=1)


def kernel(sim_mat, x):
    del x
    return (_run(sim_mat),)
